# Initial kernel scaffold; baseline (speedup 1.0000x reference)
import jax
import jax.numpy as jnp
from jax.experimental import pallas as pl


def kernel(ns_emb, adj, W, max_k):
    raise NotImplementedError("write your pallas kernel here")


# speedup vs baseline: 65.0966x; 65.0966x over previous
import functools

import jax
import jax.numpy as jnp
from jax import lax
from jax.experimental import pallas as pl
from jax.experimental.pallas import tpu as pltpu
from jax.experimental.pallas import tpu_sc as plsc

N = 4096
D = 128
K = 512
CH = 128
RB = 256
NCHUNK = N // CH
NSEG = (N * N) // CH


def _sortbits(f):
    b = lax.bitcast_convert_type(f, jnp.int32)
    return jnp.where(b >= 0, b, b ^ jnp.int32(0x7FFFFFFF))


def _better(ka, ia, kb, ib):
    return (ka > kb) | ((ka == kb) & (ia < ib))


def _cmpex(kv, ki, j, k_):
    n, C = kv.shape
    kv4 = kv.reshape(n // (2 * j), 2, j, C)
    ki4 = ki.reshape(n // (2 * j), 2, j, C)
    av, bv = kv4[:, 0], kv4[:, 1]
    ai, bi = ki4[:, 0], ki4[:, 1]
    a_bet = _better(av, ai, bv, bi)
    bet_v = jnp.where(a_bet, av, bv)
    wor_v = jnp.where(a_bet, bv, av)
    bet_i = jnp.where(a_bet, ai, bi)
    wor_i = jnp.where(a_bet, bi, ai)
    b_idx = lax.broadcasted_iota(jnp.int32, (n // (2 * j), 1, 1), 0)
    desc = ((b_idx * (2 * j)) & k_) == 0
    new_av = jnp.where(desc, bet_v, wor_v)
    new_bv = jnp.where(desc, wor_v, bet_v)
    new_ai = jnp.where(desc, bet_i, wor_i)
    new_bi = jnp.where(desc, wor_i, bet_i)
    kv = jnp.stack([new_av, new_bv], axis=1).reshape(n, C)
    ki = jnp.stack([new_ai, new_bi], axis=1).reshape(n, C)
    return kv, ki


def _colsort_desc(kv, ki):
    n = kv.shape[0]
    k_ = 2
    while k_ <= n:
        j = k_ // 2
        while j >= 1:
            kv, ki = _cmpex(kv, ki, j, k_)
            j //= 2
        k_ *= 2
    return kv, ki


def _bitonic_merge_desc(kv, ki):
    n = kv.shape[0]
    j = n // 2
    while j >= 1:
        kv, ki = _cmpex(kv, ki, j, 2 * n)
        j //= 2
    return kv, ki


def _rev0(x):
    n, C = x.shape
    j = n // 2
    while j >= 1:
        x4 = x.reshape(n // (2 * j), 2, j, C)
        x = jnp.stack([x4[:, 1], x4[:, 0]], axis=1).reshape(n, C)
        j //= 2
    return x


def _treemerge(kv, ki):
    while kv.shape[1] > 1:
        h = kv.shape[1] // 2
        av, ai = kv[:, :h], ki[:, :h]
        bv = _rev0(kv[:, h:])
        bi = _rev0(ki[:, h:])
        a_bet = _better(av, ai, bv, bi)
        kv = jnp.where(a_bet, av, bv)
        ki = jnp.where(a_bet, ai, bi)
        kv, ki = _bitonic_merge_desc(kv, ki)
    return kv[:, 0], ki[:, 0]


def _stage_a_body(ns_ref, w_ref, adj_ref, mapped_ref, segk_ref, segi_ref):
    i = pl.program_id(0)
    m = lax.dot_general(ns_ref[...], w_ref[...], (((1,), (1,)), ((), ())),
                        preferred_element_type=jnp.float32)
    mapped_ref[...] = jnp.where(m >= 0, m, jnp.float32(0.01) * m)

    row = lax.broadcasted_iota(jnp.int32, (RB, N), 0) + i * RB
    colm = lax.broadcasted_iota(jnp.int32, (RB, N), 1)
    keys = jnp.where(colm > row, _sortbits(adj_ref[...]), 0)
    k3 = keys.reshape(RB, NCHUNK, CH)
    segmax = jnp.max(k3, axis=2)
    lane = lax.broadcasted_iota(jnp.int32, (RB, NCHUNK, CH), 2)
    segarg = jnp.min(jnp.where(k3 == segmax[:, :, None], lane, CH), axis=2)
    base = row[:, :NCHUNK] * N
    chunkb = lax.broadcasted_iota(jnp.int32, (RB, NCHUNK), 1) * CH
    segk_ref[...] = segmax
    segi_ref[...] = base + chunkb + segarg


_stage_a = pl.pallas_call(
    _stage_a_body,
    grid=(N // RB,),
    in_specs=[
        pl.BlockSpec((RB, D), lambda i: (i, 0)),
        pl.BlockSpec((D, D), lambda i: (0, 0)),
        pl.BlockSpec((RB, N), lambda i: (i, 0)),
    ],
    out_specs=[
        pl.BlockSpec((RB, D), lambda i: (i, 0)),
        pl.BlockSpec((RB, NCHUNK), lambda i: (i, 0)),
        pl.BlockSpec((RB, NCHUNK), lambda i: (i, 0)),
    ],
    out_shape=[
        jax.ShapeDtypeStruct((N, D), jnp.float32),
        jax.ShapeDtypeStruct((N, NCHUNK), jnp.int32),
        jax.ShapeDtypeStruct((N, NCHUNK), jnp.int32),
    ],
)


def _stage_b_body(segk_ref, segi_ref, out_ref):
    kv, ki = _colsort_desc(segk_ref[...], segi_ref[...])
    _, topi = _treemerge(kv, ki)
    segids = topi // CH
    sk, _ = _colsort_desc(-segids[:, None], jnp.zeros((K, 1), jnp.int32))
    out_ref[...] = -sk[:, 0]


_stage_b = pl.pallas_call(
    _stage_b_body,
    out_shape=jax.ShapeDtypeStruct((K,), jnp.int32),
)


def _sc_gather(table, idx):
    info = plsc.get_sparse_core_info()
    nw = info.num_cores * info.num_subcores
    bpw = K // nw
    mesh = plsc.VectorSubcoreMesh(core_axis_name="c", subcore_axis_name="s")

    @functools.partial(
        pl.kernel,
        mesh=mesh,
        out_type=jax.ShapeDtypeStruct((K, CH), jnp.float32),
        scratch_types=[
            pltpu.VMEM((bpw,), jnp.int32),
            pltpu.VMEM((bpw, CH), jnp.float32),
            pltpu.SemaphoreType.DMA,
        ],
    )
    def gather_k(table_hbm, idx_hbm, out_hbm, idx_v, rows_v, sem):
        wid = lax.axis_index("s") * info.num_cores + lax.axis_index("c")
        base = wid * bpw
        pltpu.sync_copy(idx_hbm.at[pl.ds(base, bpw)], idx_v)
        pltpu.async_copy(table_hbm.at[idx_v], rows_v, sem).wait()
        pltpu.sync_copy(rows_v, out_hbm.at[pl.ds(base, bpw)])

    return gather_k(table, idx)


def _stage_d_body(cand_ref, segid_ref, mapped_ref, out_ref):
    sid = segid_ref[...]
    g = sid[:, None] * CH + lax.broadcasted_iota(jnp.int32, (K, CH), 1)
    r = g // N
    c = g % N
    keys = jnp.where(c > r, _sortbits(cand_ref[...]), 0)
    kv, ki = _colsort_desc(keys, g)
    _, gi = _treemerge(kv, ki)
    rows = gi // N
    cols = gi % N
    cc = lax.broadcasted_iota(jnp.int32, (K, N), 1)
    oh = ((cc == rows[:, None]).astype(jnp.float32)
          + (cc == cols[:, None]).astype(jnp.float32))
    out_ref[...] = lax.dot_general(oh, mapped_ref[...],
                                   (((1,), (0,)), ((), ())),
                                   preferred_element_type=jnp.float32)


_stage_d = pl.pallas_call(
    _stage_d_body,
    out_shape=jax.ShapeDtypeStruct((K, D), jnp.float32),
)


def kernel(ns_emb, adj, W, max_k):
    mapped, segk, segi = _stage_a(ns_emb, W, adj)
    segids = _stage_b(segk.reshape(K, NSEG // K), segi.reshape(K, NSEG // K))
    cand = _sc_gather(adj.reshape(NSEG, CH), segids)
    selected_rel = _stage_d(cand, segids, mapped)
    rel_num = (N * N - N) / 2.0
    rel_mask = jnp.arange(K) >= (rel_num + 0.0 * max_k)
    return selected_rel, rel_mask


# speedup vs baseline: 73.2274x; 1.1249x over previous
import functools

import jax
import jax.numpy as jnp
from jax import lax
from jax.experimental import pallas as pl
from jax.experimental.pallas import tpu as pltpu
from jax.experimental.pallas import tpu_sc as plsc

N = 4096
D = 128
K = 512
CH = 128
RB = 256
NCHUNK = N // CH
NSEG = (N * N) // CH


def _sortbits(f):
    b = lax.bitcast_convert_type(f, jnp.int32)
    return jnp.where(b >= 0, b, b ^ jnp.int32(0x7FFFFFFF))


def _better(ka, ia, kb, ib):
    return (ka > kb) | ((ka == kb) & (ia < ib))


def _cmpex(kv, ki, j, k_):
    n, C = kv.shape
    kv4 = kv.reshape(n // (2 * j), 2, j, C)
    ki4 = ki.reshape(n // (2 * j), 2, j, C)
    av, bv = kv4[:, 0], kv4[:, 1]
    ai, bi = ki4[:, 0], ki4[:, 1]
    a_bet = _better(av, ai, bv, bi)
    bet_v = jnp.where(a_bet, av, bv)
    wor_v = jnp.where(a_bet, bv, av)
    bet_i = jnp.where(a_bet, ai, bi)
    wor_i = jnp.where(a_bet, bi, ai)
    if k_ >= n:
        new_av, new_bv, new_ai, new_bi = bet_v, wor_v, bet_i, wor_i
    else:
        b_idx = lax.broadcasted_iota(jnp.int32, (n // (2 * j), 1, 1), 0)
        desc = ((b_idx * (2 * j)) & k_) == 0
        new_av = jnp.where(desc, bet_v, wor_v)
        new_bv = jnp.where(desc, wor_v, bet_v)
        new_ai = jnp.where(desc, bet_i, wor_i)
        new_bi = jnp.where(desc, wor_i, bet_i)
    kv = jnp.stack([new_av, new_bv], axis=1).reshape(n, C)
    ki = jnp.stack([new_ai, new_bi], axis=1).reshape(n, C)
    return kv, ki


def _colsort_desc(kv, ki):
    n = kv.shape[0]
    k_ = 2
    while k_ <= n:
        j = k_ // 2
        while j >= 1:
            kv, ki = _cmpex(kv, ki, j, k_)
            j //= 2
        k_ *= 2
    return kv, ki


def _bitonic_merge_desc(kv, ki):
    n = kv.shape[0]
    j = n // 2
    while j >= 1:
        kv, ki = _cmpex(kv, ki, j, 2 * n)
        j //= 2
    return kv, ki


def _rev0(x):
    n, C = x.shape
    j = n // 2
    while j >= 1:
        x4 = x.reshape(n // (2 * j), 2, j, C)
        x = jnp.stack([x4[:, 1], x4[:, 0]], axis=1).reshape(n, C)
        j //= 2
    return x


def _treemerge(kv, ki):
    while kv.shape[1] > 1:
        h = kv.shape[1] // 2
        av, ai = kv[:, :h], ki[:, :h]
        bv = _rev0(kv[:, h:])
        bi = _rev0(ki[:, h:])
        a_bet = _better(av, ai, bv, bi)
        kv = jnp.where(a_bet, av, bv)
        ki = jnp.where(a_bet, ai, bi)
        kv, ki = _bitonic_merge_desc(kv, ki)
    return kv[:, 0], ki[:, 0]


def _stage_a_body(ns_ref, w_ref, adj_ref, mapped_ref, segk_ref):
    i = pl.program_id(0)
    m = lax.dot_general(ns_ref[...], w_ref[...], (((1,), (1,)), ((), ())),
                        preferred_element_type=jnp.float32)
    mapped_ref[...] = jnp.where(m >= 0, m, jnp.float32(0.01) * m)

    row = lax.broadcasted_iota(jnp.int32, (RB, N), 0) + i * RB
    colm = lax.broadcasted_iota(jnp.int32, (RB, N), 1)
    keys = jnp.where(colm > row, _sortbits(adj_ref[...]), 0)
    segk_ref[...] = jnp.max(keys.reshape(RB, NCHUNK, CH), axis=2)


_stage_a = pl.pallas_call(
    _stage_a_body,
    grid=(N // RB,),
    in_specs=[
        pl.BlockSpec((RB, D), lambda i: (i, 0)),
        pl.BlockSpec((D, D), lambda i: (0, 0)),
        pl.BlockSpec((RB, N), lambda i: (i, 0)),
    ],
    out_specs=[
        pl.BlockSpec((RB, D), lambda i: (i, 0)),
        pl.BlockSpec((RB, NCHUNK), lambda i: (i, 0)),
    ],
    out_shape=[
        jax.ShapeDtypeStruct((N, D), jnp.float32),
        jax.ShapeDtypeStruct((N, NCHUNK), jnp.int32),
    ],
)


def _stage_b_body(segk_ref, out_ref):
    nc = NSEG // K
    r = lax.broadcasted_iota(jnp.int32, (K, nc), 0)
    c = lax.broadcasted_iota(jnp.int32, (K, nc), 1)
    kv, ki = _colsort_desc(segk_ref[...], r * nc + c)
    _, segids = _treemerge(kv, ki)
    sk, _ = _colsort_desc(-segids[:, None], jnp.zeros((K, 1), jnp.int32))
    out_ref[...] = -sk[:, 0]


_stage_b = pl.pallas_call(
    _stage_b_body,
    out_shape=jax.ShapeDtypeStruct((K,), jnp.int32),
)


def _sc_gather(table, idx):
    info = plsc.get_sparse_core_info()
    nw = info.num_cores * info.num_subcores
    bpw = K // nw
    mesh = plsc.VectorSubcoreMesh(core_axis_name="c", subcore_axis_name="s")

    @functools.partial(
        pl.kernel,
        mesh=mesh,
        out_type=jax.ShapeDtypeStruct((K, CH), jnp.float32),
        scratch_types=[
            pltpu.VMEM((bpw,), jnp.int32),
            pltpu.VMEM((bpw, CH), jnp.float32),
            pltpu.SemaphoreType.DMA,
        ],
    )
    def gather_k(table_hbm, idx_hbm, out_hbm, idx_v, rows_v, sem):
        wid = lax.axis_index("s") * info.num_cores + lax.axis_index("c")
        base = wid * bpw
        pltpu.sync_copy(idx_hbm.at[pl.ds(base, bpw)], idx_v)
        pltpu.async_copy(table_hbm.at[idx_v], rows_v, sem).wait()
        pltpu.sync_copy(rows_v, out_hbm.at[pl.ds(base, bpw)])

    return gather_k(table, idx)


def _stage_d_body(cand_ref, segid_ref, mapped_ref, out_ref):
    sid = segid_ref[...]
    g = sid[:, None] * CH + lax.broadcasted_iota(jnp.int32, (K, CH), 1)
    r = g // N
    c = g % N
    keys = jnp.where(c > r, _sortbits(cand_ref[...]), 0)
    kv, ki = _colsort_desc(keys, g)
    _, gi = _treemerge(kv, ki)
    rows = gi // N
    cols = gi % N
    cc = lax.broadcasted_iota(jnp.int32, (K, N), 1)
    oh = ((cc == rows[:, None]).astype(jnp.float32)
          + (cc == cols[:, None]).astype(jnp.float32))
    out_ref[...] = lax.dot_general(oh, mapped_ref[...],
                                   (((1,), (0,)), ((), ())),
                                   preferred_element_type=jnp.float32)


_stage_d = pl.pallas_call(
    _stage_d_body,
    out_shape=jax.ShapeDtypeStruct((K, D), jnp.float32),
)


def kernel(ns_emb, adj, W, max_k):
    mapped, segk = _stage_a(ns_emb, W, adj)
    segids = _stage_b(segk.reshape(K, NSEG // K))
    cand = _sc_gather(adj.reshape(NSEG, CH), segids)
    selected_rel = _stage_d(cand, segids, mapped)
    rel_num = (N * N - N) / 2.0
    rel_mask = jnp.arange(K) >= (rel_num + 0.0 * max_k)
    return selected_rel, rel_mask
